# speedup vs baseline: 10.0785x; 10.0785x over previous
import functools

import jax
import jax.numpy as jnp
from jax import lax
from jax.experimental import pallas as pl
from jax.experimental.pallas import tpu as pltpu
from jax.experimental.pallas import tpu_sc as plsc

N = 10000
E = 320000
D = 128

NC = 2
NS = 16
NW = NC * NS

C = 128
EPT = -(-E // NW // C) * C
E_PAD = EPT * NW
NCHUNK = EPT // C

N_PAD = 10240
RPT = N_PAD // NS

_mesh = plsc.VectorSubcoreMesh(core_axis_name="c", subcore_axis_name="s",
                               num_cores=NC, num_subcores=NS)


@functools.partial(
    pl.kernel,
    out_type=jax.ShapeDtypeStruct((NC, N_PAD), jnp.float32),
    mesh=_mesh,
    scratch_types=[
        pltpu.VMEM((C,), jnp.int32),
        pltpu.VMEM((C,), jnp.float32),
        pltpu.VMEM((128,), jnp.float32),
        pltpu.VMEM_SHARED((N_PAD,), jnp.float32),
    ],
)
def _sc_degree(dst_hbm, out_hbm, didx, ones, zbuf, accum):
    c = lax.axis_index("c")
    s = lax.axis_index("s")
    wid = s * NC + c

    for j in range(8):
        zbuf[pl.ds(j * 16, 16)] = jnp.zeros((16,), jnp.float32)
        ones[pl.ds(j * 16, 16)] = jnp.ones((16,), jnp.float32)

    base = s * RPT

    def zloop(t, carry):
        pltpu.sync_copy(zbuf, accum.at[pl.ds(base + t * 128, 128)])
        return carry

    lax.fori_loop(0, RPT // 128, zloop, 0)
    plsc.subcore_barrier()

    eb = wid * EPT

    def eloop(t, carry):
        pltpu.sync_copy(dst_hbm.at[pl.ds(eb + t * C, C)], didx)
        pltpu.sync_copy(ones, accum.at[didx], add=True)
        return carry

    lax.fori_loop(0, NCHUNK, eloop, 0)
    plsc.subcore_barrier()

    pltpu.sync_copy(accum.at[pl.ds(base, RPT)], out_hbm.at[c, pl.ds(base, RPT)])


@functools.partial(
    pl.kernel,
    out_type=jax.ShapeDtypeStruct((NC, N_PAD, D), jnp.float32),
    mesh=_mesh,
    scratch_types=[
        pltpu.VMEM((C,), jnp.int32),
        pltpu.VMEM((C,), jnp.int32),
        pltpu.VMEM((C, D), jnp.float32),
        pltpu.VMEM((16, D), jnp.float32),
        pltpu.VMEM_SHARED((N_PAD, D), jnp.float32),
        pltpu.SemaphoreType.DMA,
    ],
)
def _sc_scatter(g_hbm, src_hbm, dst_hbm, out_hbm, sidx, didx, rows, zbuf,
                accum, sem):
    c = lax.axis_index("c")
    s = lax.axis_index("s")
    wid = s * NC + c

    for i in range(16):
        for j in range(8):
            zbuf[i, pl.ds(j * 16, 16)] = jnp.zeros((16,), jnp.float32)

    base = s * RPT

    def zloop(t, carry):
        pltpu.sync_copy(zbuf, accum.at[pl.ds(base + t * 16, 16)])
        return carry

    lax.fori_loop(0, RPT // 16, zloop, 0)
    plsc.subcore_barrier()

    eb = wid * EPT

    def eloop(t, carry):
        off = eb + t * C
        pltpu.sync_copy(src_hbm.at[pl.ds(off, C)], sidx)
        cp = pltpu.async_copy(g_hbm.at[sidx], rows, sem)
        pltpu.sync_copy(dst_hbm.at[pl.ds(off, C)], didx)
        cp.wait()
        pltpu.sync_copy(rows, accum.at[didx], add=True)
        return carry

    lax.fori_loop(0, NCHUNK, eloop, 0)
    plsc.subcore_barrier()

    pltpu.sync_copy(accum.at[pl.ds(base, RPT)],
                    out_hbm.at[c, pl.ds(base, RPT)])


BN = 2000


def _dinv(d0, d1):
    return lax.rsqrt(d0 + d1 + 1.0)


def _tc_pre_body(d0_ref, d1_ref, x_ref, w_ref, o_ref):
    dinv = _dinv(d0_ref[...], d1_ref[...])
    o_ref[...] = dinv * jnp.dot(x_ref[...], w_ref[...],
                                preferred_element_type=jnp.float32)


def _tc_mid_body(p0_ref, p1_ref, g_ref, d0_ref, d1_ref, b_ref, w_ref, o_ref):
    dinv = _dinv(d0_ref[...], d1_ref[...])
    h = dinv * (p0_ref[...] + p1_ref[...] + g_ref[...]) + b_ref[...]
    h = jnp.maximum(h, 0.0)
    o_ref[...] = dinv * jnp.dot(h, w_ref[...],
                                preferred_element_type=jnp.float32)


def _tc_fin_body(p0_ref, p1_ref, g_ref, d0_ref, d1_ref, b_ref, o_ref):
    dinv = _dinv(d0_ref[...], d1_ref[...])
    o_ref[...] = dinv * (p0_ref[...] + p1_ref[...] + g_ref[...]) + b_ref[...]


_row_spec = pl.BlockSpec((BN, D), lambda i: (i, 0))
_col_spec = pl.BlockSpec((BN, 1), lambda i: (i, 0))
_w_spec = pl.BlockSpec((D, D), lambda i: (0, 0))
_b_spec = pl.BlockSpec((1, D), lambda i: (0, 0))
_grid = (N // BN,)
_out_sds = jax.ShapeDtypeStruct((N, D), jnp.float32)

_tc_pre = pl.pallas_call(
    _tc_pre_body, grid=_grid,
    in_specs=[_col_spec, _col_spec, _row_spec, _w_spec],
    out_specs=_row_spec, out_shape=_out_sds)

_tc_mid = pl.pallas_call(
    _tc_mid_body, grid=_grid,
    in_specs=[_row_spec, _row_spec, _row_spec, _col_spec, _col_spec,
              _b_spec, _w_spec],
    out_specs=_row_spec, out_shape=_out_sds)

_tc_fin = pl.pallas_call(
    _tc_fin_body, grid=_grid,
    in_specs=[_row_spec, _row_spec, _row_spec, _col_spec, _col_spec, _b_spec],
    out_specs=_row_spec, out_shape=_out_sds)


def kernel(x, edge_index, W1, b1, W2, b2, W3, b3):
    src = edge_index[0].astype(jnp.int32)
    dst = edge_index[1].astype(jnp.int32)
    pad = E_PAD - E
    src_p = jnp.concatenate([src, jnp.zeros((pad,), jnp.int32)])
    dst_p = jnp.concatenate([dst, jnp.full((pad,), N_PAD - 1, jnp.int32)])

    degp = _sc_degree(dst_p)
    d0 = degp[0, :N, None]
    d1 = degp[1, :N, None]

    b1r = b1.reshape(1, D)
    b2r = b2.reshape(1, D)
    b3r = b3.reshape(1, D)

    g1 = _tc_pre(d0, d1, x, W1)
    p = _sc_scatter(g1, src_p, dst_p)
    g2 = _tc_mid(p[0, :N], p[1, :N], g1, d0, d1, b1r, W2)
    p = _sc_scatter(g2, src_p, dst_p)
    g3 = _tc_mid(p[0, :N], p[1, :N], g2, d0, d1, b2r, W3)
    p = _sc_scatter(g3, src_p, dst_p)
    return _tc_fin(p[0, :N], p[1, :N], g3, d0, d1, b3r)
